# speedup vs baseline: 4.0808x; 4.0808x over previous
import functools

import jax
import jax.numpy as jnp
from jax import lax
from jax.experimental import pallas as pl
from jax.experimental.pallas import tpu as pltpu
from jax.experimental.pallas import tpu_sc as plsc

VOCAB = 50304
DIM = 768
NEMB = 6
BATCH = 4
SEQ = 2048

NW = 32
ROWS = BATCH * SEQ
RPW = ROWS // NW
CHUNK = 64
NCHUNK = RPW // CHUNK
NSTEPS = NEMB * NCHUNK

_mesh = plsc.VectorSubcoreMesh(core_axis_name="c", subcore_axis_name="s")


@functools.partial(
    pl.kernel,
    mesh=_mesh,
    out_type=[jax.ShapeDtypeStruct((ROWS, DIM), jnp.float32)
              for _ in range(NEMB)],
    scratch_types=[
        pltpu.VMEM((NEMB, RPW), jnp.int32),
        pltpu.VMEM((CHUNK, DIM), jnp.float32),
        pltpu.VMEM((CHUNK, DIM), jnp.float32),
        pltpu.SemaphoreType.DMA,
        pltpu.SemaphoreType.DMA,
        pltpu.SemaphoreType.DMA,
        pltpu.SemaphoreType.DMA,
    ],
)
def _gather6(idx_hbm, tab_hbm, o0, o1, o2, o3, o4, o5,
             idx_v, buf0, buf1, g0, g1, w0, w1):
    outs = (o0, o1, o2, o3, o4, o5)
    bufs = (buf0, buf1)
    gsems = (g0, g1)
    wsems = (w0, w1)
    wid = lax.axis_index("s") * 2 + lax.axis_index("c")
    base = wid * RPW
    pltpu.sync_copy(idx_hbm.at[wid], idx_v)

    def gather(step):
        t, ch = divmod(step, NCHUNK)
        b = step % 2
        return pltpu.async_copy(
            tab_hbm.at[idx_v.at[t, pl.ds(ch * CHUNK, CHUNK)]],
            bufs[b], gsems[b])

    def write(step):
        t, ch = divmod(step, NCHUNK)
        b = step % 2
        return pltpu.async_copy(
            bufs[b], outs[t].at[pl.ds(base + ch * CHUNK, CHUNK)], wsems[b])

    writes = [None] * NSTEPS
    g = gather(0)
    for s in range(NSTEPS):
        g.wait()
        writes[s] = write(s)
        if s + 1 < NSTEPS:
            if s >= 1:
                writes[s - 1].wait()
            g = gather(s + 1)
    writes[NSTEPS - 2].wait()
    writes[NSTEPS - 1].wait()


def kernel(inputs, tables):
    flat = inputs.reshape(-1).astype(jnp.int32)
    offs = (jnp.arange(NEMB, dtype=jnp.int32) * VOCAB)[:, None]
    idx_all = (flat[None, :] + offs).reshape(NEMB, NW, RPW).transpose(1, 0, 2)
    tab = tables.reshape(NEMB * VOCAB, DIM)
    outs = _gather6(idx_all, tab)
    ve = [o.reshape(BATCH, SEQ, DIM) for o in outs]
    return tuple(ve + ve[::-1])


# speedup vs baseline: 4.8554x; 1.1898x over previous
import functools

import jax
import jax.numpy as jnp
from jax import lax
from jax.experimental import pallas as pl
from jax.experimental.pallas import tpu as pltpu
from jax.experimental.pallas import tpu_sc as plsc

VOCAB = 50304
DIM = 768
NEMB = 6
BATCH = 4
SEQ = 2048

NW = 32
ROWS = BATCH * SEQ
RPW = ROWS // NW
CHUNK = 64
NCHUNK = RPW // CHUNK
NSTEPS = NEMB * NCHUNK

_mesh = plsc.VectorSubcoreMesh(core_axis_name="c", subcore_axis_name="s")


@functools.partial(
    pl.kernel,
    mesh=_mesh,
    out_type=[jax.ShapeDtypeStruct((ROWS, DIM), jnp.float32)
              for _ in range(2 * NEMB)],
    scratch_types=[
        pltpu.VMEM((NEMB, RPW), jnp.int32),
        pltpu.VMEM((CHUNK, DIM), jnp.float32),
        pltpu.VMEM((CHUNK, DIM), jnp.float32),
        pltpu.SemaphoreType.DMA,
        pltpu.SemaphoreType.DMA,
        pltpu.SemaphoreType.DMA,
        pltpu.SemaphoreType.DMA,
    ],
)
def _gather6(idx_hbm, tab_hbm,
             o0, o1, o2, o3, o4, o5, o6, o7, o8, o9, o10, o11,
             idx_v, buf0, buf1, g0, g1, w0, w1):
    outs = (o0, o1, o2, o3, o4, o5, o6, o7, o8, o9, o10, o11)
    bufs = (buf0, buf1)
    gsems = (g0, g1)
    wsems = (w0, w1)
    wid = lax.axis_index("s") * 2 + lax.axis_index("c")
    base = wid * RPW
    pltpu.sync_copy(idx_hbm.at[wid], idx_v)

    def gather(step):
        t, ch = divmod(step, NCHUNK)
        b = step % 2
        return pltpu.async_copy(
            tab_hbm.at[idx_v.at[t, pl.ds(ch * CHUNK, CHUNK)]],
            bufs[b], gsems[b])

    def write(step):
        t, ch = divmod(step, NCHUNK)
        b = step % 2
        dst = pl.ds(base + ch * CHUNK, CHUNK)
        w1_ = pltpu.async_copy(bufs[b], outs[t].at[dst], wsems[b])
        w2_ = pltpu.async_copy(bufs[b], outs[11 - t].at[dst], wsems[b])
        return (w1_, w2_)

    writes = [None] * NSTEPS
    g = gather(0)
    for s in range(NSTEPS):
        g.wait()
        writes[s] = write(s)
        if s + 1 < NSTEPS:
            if s >= 1:
                for w in writes[s - 1]:
                    w.wait()
            g = gather(s + 1)
    for s in (NSTEPS - 2, NSTEPS - 1):
        for w in writes[s]:
            w.wait()


def kernel(inputs, tables):
    flat = inputs.reshape(-1).astype(jnp.int32)
    offs = (jnp.arange(NEMB, dtype=jnp.int32) * VOCAB)[:, None]
    idx_all = (flat[None, :] + offs).reshape(NEMB, NW, RPW).transpose(1, 0, 2)
    tab = tables.reshape(NEMB * VOCAB, DIM)
    outs = _gather6(idx_all, tab)
    return tuple(o.reshape(BATCH, SEQ, DIM) for o in outs)
